# Initial kernel scaffold; baseline (speedup 1.0000x reference)
import jax
import jax.numpy as jnp
from jax.experimental import pallas as pl


def kernel(vecs, codebook):
    raise NotImplementedError("write your pallas kernel here")


# speedup vs baseline: 3.4886x; 3.4886x over previous
import functools

import jax
import jax.numpy as jnp
from jax.experimental import pallas as pl
from jax.experimental.pallas import tpu as pltpu

BATCH = 1024
EMBED = 768
PARTITION = 96
CENTROIDS = 256
DSUB = 8
GROUP = 16
NGROUPS = PARTITION // GROUP
SEG = GROUP * CENTROIDS


def _quant_group(vec_ref, cbt_ref, cb_ref, out_ref, w_ref, c_ref):
    w_ref[...] = jnp.zeros((GROUP * DSUB, SEG), jnp.float32)
    c_ref[...] = jnp.zeros((SEG, GROUP * DSUB), jnp.float32)
    for q in range(GROUP):
        w_ref[q * DSUB:(q + 1) * DSUB, q * CENTROIDS:(q + 1) * CENTROIDS] = cbt_ref[q]
        c_ref[q * CENTROIDS:(q + 1) * CENTROIDS, q * DSUB:(q + 1) * DSUB] = cb_ref[q]

    w = w_ref[...]
    cnorm = jnp.sum(w * w, axis=0, keepdims=True)
    v = vec_ref[...]
    scores = jax.lax.dot_general(
        v, w, (((1,), (0,)), ((), ())),
        precision=jax.lax.Precision.HIGHEST,
        preferred_element_type=jnp.float32)
    adj = 2.0 * scores - cnorm

    hots = []
    for q in range(GROUP):
        seg = adj[:, q * CENTROIDS:(q + 1) * CENTROIDS]
        m = jnp.max(seg, axis=1, keepdims=True)
        iota = jax.lax.broadcasted_iota(jnp.int32, seg.shape, 1)
        cand = jnp.where(seg == m, iota, CENTROIDS)
        idx = jnp.min(cand, axis=1, keepdims=True)
        hots.append((iota == idx).astype(jnp.float32))
    hot = jnp.concatenate(hots, axis=1)
    out_ref[...] = jax.lax.dot_general(
        hot, c_ref[...], (((1,), (0,)), ((), ())),
        precision=jax.lax.Precision.HIGHEST,
        preferred_element_type=jnp.float32)


@jax.jit
def kernel(vecs, codebook):
    cbt = codebook.transpose(0, 2, 1)
    return pl.pallas_call(
        _quant_group,
        grid=(NGROUPS,),
        in_specs=[
            pl.BlockSpec((BATCH, GROUP * DSUB), lambda g: (0, g)),
            pl.BlockSpec((GROUP, DSUB, CENTROIDS), lambda g: (g, 0, 0)),
            pl.BlockSpec((GROUP, CENTROIDS, DSUB), lambda g: (g, 0, 0)),
        ],
        out_specs=pl.BlockSpec((BATCH, GROUP * DSUB), lambda g: (0, g)),
        out_shape=jax.ShapeDtypeStruct((BATCH, EMBED), jnp.float32),
        scratch_shapes=[
            pltpu.VMEM((GROUP * DSUB, SEG), jnp.float32),
            pltpu.VMEM((SEG, GROUP * DSUB), jnp.float32),
        ],
    )(vecs, cbt, codebook)
